# Initial kernel scaffold; baseline (speedup 1.0000x reference)
import jax
import jax.numpy as jnp
from jax.experimental import pallas as pl


def kernel(x, W1, b1, W2, b2, Wg, bg):
    raise NotImplementedError("write your pallas kernel here")


# speedup vs baseline: 1.2091x; 1.2091x over previous
import functools

import jax
import jax.numpy as jnp
from jax.experimental import pallas as pl
from jax.experimental.pallas import tpu as pltpu


def _moe_dense_kernel(x_ref, wg_ref, bg_ref, w1_ref, b1_ref, w2_ref, b2_ref,
                      out_ref, ent_ref, gates_ref, *, n_tokens):
    t = pl.program_id(0)
    e = pl.program_id(1)

    @pl.when(e == 0)
    def _router():
        x = x_ref[...]
        logits = jnp.dot(x, wg_ref[...], preferred_element_type=jnp.float32)
        logits = logits + bg_ref[...]
        bt, ne = logits.shape
        idx = jax.lax.broadcasted_iota(jnp.int32, (bt, ne), 1)
        m1 = jnp.max(logits, axis=1, keepdims=True)
        is1 = logits == m1
        i1 = jnp.min(jnp.where(is1, idx, ne), axis=1, keepdims=True)
        oh1 = idx == i1
        masked = jnp.where(oh1, -jnp.inf, logits)
        m2 = jnp.max(masked, axis=1, keepdims=True)
        is2 = masked == m2
        i2 = jnp.min(jnp.where(is2, idx, ne), axis=1, keepdims=True)
        oh2 = idx == i2
        z = jnp.exp(m2 - m1)
        denom = 1.0 + z
        g1 = 1.0 / denom
        g2 = z / denom
        gates_ref[...] = g1 * oh1.astype(jnp.float32) + g2 * oh2.astype(jnp.float32)
        ent_tok = -(g1 * jnp.log(jnp.clip(g1, 1e-8, None))
                    + g2 * jnp.log(jnp.clip(g2, 1e-8, None)))
        part = (jnp.sum(ent_tok) / n_tokens).reshape(1, 1)
        prev = jnp.where(t == 0, jnp.zeros((1, 1), jnp.float32), ent_ref[...])
        ent_ref[...] = prev + part

    ne = gates_ref.shape[1]
    lane = jax.lax.broadcasted_iota(jnp.int32, (1, ne), 1)
    w = jnp.sum(gates_ref[...] * (lane == e).astype(jnp.float32),
                axis=1, keepdims=True)
    x = x_ref[...]
    h = jnp.dot(x, w1_ref[0], preferred_element_type=jnp.float32)
    h = jnp.maximum(h + b1_ref[0], 0.0)
    y = jnp.dot(h, w2_ref[0], preferred_element_type=jnp.float32)
    y = y + b2_ref[0]
    contrib = y * w

    @pl.when(e == 0)
    def _init():
        out_ref[...] = contrib

    @pl.when(e > 0)
    def _acc():
        out_ref[...] += contrib


def kernel(x, W1, b1, W2, b2, Wg, bg):
    B, N, D = x.shape
    E, _, DFF = W1.shape
    xf = x.reshape(N, D)
    BT = 512
    T = N // BT

    kern = functools.partial(_moe_dense_kernel, n_tokens=N)
    out, ent = pl.pallas_call(
        kern,
        grid=(T, E),
        in_specs=[
            pl.BlockSpec((BT, D), lambda t, e: (t, 0)),
            pl.BlockSpec((D, E), lambda t, e: (0, 0)),
            pl.BlockSpec((1, E), lambda t, e: (0, 0)),
            pl.BlockSpec((1, D, DFF), lambda t, e: (e, 0, 0)),
            pl.BlockSpec((1, 1, DFF), lambda t, e: (e, 0, 0)),
            pl.BlockSpec((1, DFF, D), lambda t, e: (e, 0, 0)),
            pl.BlockSpec((1, 1, D), lambda t, e: (e, 0, 0)),
        ],
        out_specs=[
            pl.BlockSpec((BT, D), lambda t, e: (t, 0)),
            pl.BlockSpec((1, 1), lambda t, e: (0, 0)),
        ],
        out_shape=[
            jax.ShapeDtypeStruct((N, D), jnp.float32),
            jax.ShapeDtypeStruct((1, 1), jnp.float32),
        ],
        scratch_shapes=[pltpu.VMEM((BT, E), jnp.float32)],
    )(xf, Wg, bg.reshape(1, E), W1, b1.reshape(E, 1, DFF),
      W2, b2.reshape(E, 1, D))

    return out.reshape(B, N, D), ent[0, 0]
